# baseline (speedup 1.0000x reference)
import functools

import jax
import jax.numpy as jnp
from jax import lax
from jax.experimental import pallas as pl
from jax.experimental.pallas import tpu as pltpu
from jax.experimental.pallas import tpu_sc as plsc

N_NODES = 10000
N_EDGES = 320000
H = 128
D_INV = 16

NC = 2
NS = 16
NW = NC * NS

EPW = N_EDGES // NW
CHUNK = 80
NCHUNK = EPW // CHUNK
STRIPE = 640
ZROWS = 40

@functools.cache
def _sc_mesh():
    return plsc.VectorSubcoreMesh(
        core_axis_name="c", subcore_axis_name="s", num_cores=NC, num_subcores=NS
    )


def _silu(v):
    return v * (1.0 / (1.0 + jnp.exp(-v)))


def _pack16(v):
    lo = jax.lax.bitcast_convert_type(v[:, : H // 2].astype(jnp.bfloat16), jnp.uint16)
    hi = jax.lax.bitcast_convert_type(v[:, H // 2 :].astype(jnp.bfloat16), jnp.uint16)
    w = lo.astype(jnp.uint32) | (hi.astype(jnp.uint32) << 16)
    return w.astype(jnp.int32)


def _pq_body(x_ref, wa_ref, wb_ref, b_ref, p_ref, q_ref):
    x = x_ref[...]
    p_ref[...] = jnp.dot(x, wa_ref[...], preferred_element_type=jnp.float32) + b_ref[...]
    q_ref[...] = jnp.dot(x, wb_ref[...], preferred_element_type=jnp.float32)


def _pq(x, wa, wb, b):
    bn = 2000
    grid = (N_NODES // bn,)
    return pl.pallas_call(
        _pq_body,
        grid=grid,
        in_specs=[
            pl.BlockSpec((bn, H), lambda i: (i, 0)),
            pl.BlockSpec((H, H), lambda i: (0, 0)),
            pl.BlockSpec((H, H), lambda i: (0, 0)),
            pl.BlockSpec((1, H), lambda i: (0, 0)),
        ],
        out_specs=[
            pl.BlockSpec((bn, H), lambda i: (i, 0)),
            pl.BlockSpec((bn, H), lambda i: (i, 0)),
        ],
        out_shape=[
            jax.ShapeDtypeStruct((N_NODES, H), jnp.float32),
            jax.ShapeDtypeStruct((N_NODES, H), jnp.float32),
        ],
    )(x, wa, wb, b)


@functools.cache
def _gather_add_kernel():
    return pl.kernel(
        _gather_add_body,
        out_type=jax.ShapeDtypeStruct((N_EDGES, H // 2), jnp.int32),
        mesh=_sc_mesh(),
        scratch_types=[
            pltpu.VMEM((NCHUNK, CHUNK), jnp.int32),
            pltpu.VMEM((NCHUNK, CHUNK), jnp.int32),
            pltpu.VMEM((3, CHUNK, H), jnp.float32),
            pltpu.VMEM((3, CHUNK, H), jnp.float32),
            pltpu.VMEM((2, CHUNK, H // 2), jnp.int32),
            pltpu.SemaphoreType.DMA,
            pltpu.SemaphoreType.DMA,
            pltpu.SemaphoreType.DMA,
            pltpu.SemaphoreType.DMA,
            pltpu.SemaphoreType.DMA,
        ],
    )


def _gather_add_body(
    p_hbm, q_hbm, dst_hbm, src_hbm, t_hbm,
    idx_d, idx_s, bufp, bufq, bufo, semg0, semg1, semg2, sems0, sems1,
):
    wid = lax.axis_index("s") * NC + lax.axis_index("c")
    wbase = wid * EPW
    pltpu.sync_copy(dst_hbm.at[wid], idx_d)
    pltpu.sync_copy(src_hbm.at[wid], idx_s)
    semg = (semg0, semg1, semg2)
    sems = (sems0, sems1)

    def g_issue(c, slot):
        pltpu.async_copy(p_hbm.at[idx_d.at[c]], bufp.at[slot], semg[slot])
        pltpu.async_copy(q_hbm.at[idx_s.at[c]], bufq.at[slot], semg[slot])

    def g_wait(slot):
        pltpu.make_async_copy(p_hbm.at[idx_d.at[0]], bufp.at[slot], semg[slot]).wait()
        pltpu.make_async_copy(q_hbm.at[idx_s.at[0]], bufq.at[slot], semg[slot]).wait()

    def s_issue(c, oslot):
        base = pl.multiple_of(wbase + c * CHUNK, 8)
        pltpu.async_copy(bufo.at[oslot], t_hbm.at[pl.ds(base, CHUNK)], sems[oslot])

    def s_wait(oslot):
        pltpu.make_async_copy(bufo.at[oslot], t_hbm.at[pl.ds(0, CHUNK)], sems[oslot]).wait()

    maskhi = jnp.full((16,), 0xFFFF0000, jnp.uint32)

    def add(slot, oslot):
        def addrow(r, carry):
            bc = jax.lax.bitcast_convert_type
            for cc in range(H // 32):
                sl_lo = pl.ds(cc * 16, 16)
                sl_hi = pl.ds(H // 2 + cc * 16, 16)
                lo = bufp[slot, r, sl_lo] + bufq[slot, r, sl_lo]
                hi = bufp[slot, r, sl_hi] + bufq[slot, r, sl_hi]
                w = (bc(lo, jnp.uint32) >> 16) | (bc(hi, jnp.uint32) & maskhi)
                bufo[oslot, r, sl_lo] = bc(w, jnp.int32)
            return carry

        lax.fori_loop(0, CHUNK, addrow, 0, unroll=2)

    def substep(c, slot, oslot, issue_next, wait_store):
        if issue_next:
            g_issue(c + 2, (slot + 2) % 3)
        g_wait(slot)
        if wait_store is None:
            s_wait(oslot)
        elif wait_store is not None and wait_store is not False:
            @pl.when(wait_store)
            def _():
                s_wait(oslot)
        add(slot, oslot)
        s_issue(c, oslot)

    g_issue(0, 0)
    g_issue(1, 1)

    def group(i, carry):
        c0 = i * 6
        for j in range(6):
            ws = (i > 0) if j < 2 else None
            substep(c0 + j, j % 3, j % 2, True, ws)
        return carry

    lax.fori_loop(0, (NCHUNK - 5) // 6, group, 0)
    for c in range(NCHUNK - 5, NCHUNK):
        substep(c, c % 3, c % 2, c + 2 < NCHUNK, None)
    s_wait((NCHUNK - 2) % 2)
    s_wait((NCHUNK - 1) % 2)


def _msg_body(t_ref, inv_ref, wc_ref, w2_ref, b2_ref, m_ref):
    w = t_ref[...]
    t_lo = jax.lax.bitcast_convert_type(w << 16, jnp.float32)
    t_hi = jax.lax.bitcast_convert_type(
        jnp.bitwise_and(w, jnp.int32(-65536)), jnp.float32
    )
    inv = inv_ref[...]
    h_lo = _silu(
        t_lo + jnp.dot(inv, wc_ref[:, : H // 2], preferred_element_type=jnp.float32)
    )
    h_hi = _silu(
        t_hi + jnp.dot(inv, wc_ref[:, H // 2 :], preferred_element_type=jnp.float32)
    )
    m = (
        jnp.dot(h_lo, w2_ref[: H // 2], preferred_element_type=jnp.float32)
        + jnp.dot(h_hi, w2_ref[H // 2 :], preferred_element_type=jnp.float32)
        + b2_ref[...]
    )
    m_ref[...] = _pack16(_silu(m))


def _msg(t, inv, wc, w2, b2):
    be = 4000
    grid = (N_EDGES // be,)
    return pl.pallas_call(
        _msg_body,
        grid=grid,
        in_specs=[
            pl.BlockSpec((be, H // 2), lambda i: (i, 0)),
            pl.BlockSpec((be, D_INV), lambda i: (i, 0)),
            pl.BlockSpec((D_INV, H), lambda i: (0, 0)),
            pl.BlockSpec((H, H), lambda i: (0, 0)),
            pl.BlockSpec((1, H), lambda i: (0, 0)),
        ],
        out_specs=pl.BlockSpec((be, H // 2), lambda i: (i, 0)),
        out_shape=jax.ShapeDtypeStruct((N_EDGES, H // 2), jnp.int32),
    )(t, inv, wc, w2, b2)


@functools.cache
def _scatter_add_kernel():
    return pl.kernel(
        _scatter_add_body,
        out_type=jax.ShapeDtypeStruct((NC, N_NODES, H), jnp.float32),
        mesh=_sc_mesh(),
        scratch_types=[
            pltpu.VMEM((4, CHUNK), jnp.int32),
            pltpu.VMEM((2, CHUNK, H // 2), jnp.int32),
            pltpu.VMEM((2, CHUNK, H), jnp.float32),
            pltpu.VMEM((ZROWS, H), jnp.float32),
            pltpu.VMEM_SHARED((N_NODES, H), jnp.float32),
            pltpu.SemaphoreType.DMA,
            pltpu.SemaphoreType.DMA,
            pltpu.SemaphoreType.DMA,
            pltpu.SemaphoreType.DMA,
        ],
    )


def _scatter_add_body(
    m_hbm, dst_hbm, out_hbm, idx_d, mbuf, fbuf, zbuf, agg_sh,
    seml0, seml1, sema0, sema1,
):
    cid = lax.axis_index("c")
    sid = lax.axis_index("s")
    wid = sid * NC + cid
    wbase = wid * EPW

    zero = jnp.zeros((16,), jnp.float32)

    def zrow(r, carry):
        for cc in range(H // 16):
            zbuf[r, pl.ds(cc * 16, 16)] = zero
        return carry

    lax.fori_loop(0, ZROWS, zrow, 0)
    nstripe = jnp.where(sid == NS - 1, (N_NODES - (NS - 1) * STRIPE) // ZROWS, STRIPE // ZROWS)

    def zcopy(j, carry):
        pltpu.sync_copy(zbuf, agg_sh.at[pl.ds(pl.multiple_of(sid * STRIPE + j * ZROWS, 8), ZROWS)])
        return carry

    lax.fori_loop(0, nstripe, zcopy, 0)
    plsc.subcore_barrier()

    seml = (seml0, seml1)
    sema = (sema0, sema1)
    maskhi = jnp.full((16,), -65536, jnp.int32)

    def m_issue(c, slot, islot):
        base = pl.multiple_of(wbase + c * CHUNK, 8)
        pltpu.async_copy(m_hbm.at[pl.ds(base, CHUNK)], mbuf.at[slot], seml[slot])
        pltpu.async_copy(dst_hbm.at[wid, c], idx_d.at[islot], seml[slot])

    def m_wait(slot):
        pltpu.make_async_copy(m_hbm.at[pl.ds(0, CHUNK)], mbuf.at[slot], seml[slot]).wait()
        pltpu.make_async_copy(dst_hbm.at[wid, 0], idx_d.at[0], seml[slot]).wait()

    def a_issue(c, oslot, islot):
        pltpu.async_copy(fbuf.at[oslot], agg_sh.at[idx_d.at[islot]], sema[oslot], add=True)

    def a_wait(oslot):
        pltpu.make_async_copy(fbuf.at[oslot], agg_sh.at[idx_d.at[0]], sema[oslot]).wait()

    def unpack(slot, oslot):
        def uprow(r, carry):
            bc = jax.lax.bitcast_convert_type
            for cc in range(H // 32):
                sl = pl.ds(cc * 16, 16)
                w = mbuf[slot, r, sl]
                fbuf[oslot, r, sl] = bc(w << 16, jnp.float32)
                fbuf[oslot, r, pl.ds(H // 2 + cc * 16, 16)] = bc(w & maskhi, jnp.float32)
            return carry

        lax.fori_loop(0, CHUNK, uprow, 0, unroll=2)

    def substep(g, c, j):
        m_wait(j % 2)
        if j >= 2:
            a_wait(j % 2)
        else:
            @pl.when(g > 0)
            def _():
                a_wait(j % 2)
        unpack(j % 2, j % 2)
        a_issue(c, j % 2, j % 4)

        @pl.when(c + 2 < NCHUNK)
        def _():
            m_issue(c + 2, j % 2, (j + 2) % 4)

    m_issue(0, 0, 0)
    m_issue(1, 1, 1)

    def group(g, carry):
        c0 = g * 4
        for j in range(4):
            substep(g, c0 + j, j)
        return carry

    lax.fori_loop(0, (NCHUNK - 1) // 4, group, 0)
    m_wait(0)
    a_wait(0)
    unpack(0, 0)
    a_issue(NCHUNK - 1, 0, 0)
    a_wait(1)
    a_wait(0)
    plsc.subcore_barrier()

    def ocopy(j, carry):
        r0 = pl.multiple_of(sid * STRIPE + j * ZROWS, 8)
        pltpu.sync_copy(agg_sh.at[pl.ds(r0, ZROWS)], zbuf)
        pltpu.sync_copy(zbuf, out_hbm.at[cid, pl.ds(r0, ZROWS)])
        return carry

    lax.fori_loop(0, nstripe, ocopy, 0)


def _upd_body(x_ref, agg_ref, wa_ref, wb_ref, b1_ref, w2_ref, b2_ref, o_ref):
    x = x_ref[...]
    agg = agg_ref[0] + agg_ref[1]
    u = _silu(
        jnp.dot(x, wa_ref[...], preferred_element_type=jnp.float32)
        + jnp.dot(agg, wb_ref[...], preferred_element_type=jnp.float32)
        + b1_ref[...]
    )
    o_ref[...] = x + jnp.dot(u, w2_ref[...], preferred_element_type=jnp.float32) + b2_ref[...]


def _upd(x, aggs, wa, wb, b1, w2, b2):
    bn = 2000
    grid = (N_NODES // bn,)
    return pl.pallas_call(
        _upd_body,
        grid=grid,
        in_specs=[
            pl.BlockSpec((bn, H), lambda i: (i, 0)),
            pl.BlockSpec((NC, bn, H), lambda i: (0, i, 0)),
            pl.BlockSpec((H, H), lambda i: (0, 0)),
            pl.BlockSpec((H, H), lambda i: (0, 0)),
            pl.BlockSpec((1, H), lambda i: (0, 0)),
            pl.BlockSpec((H, H), lambda i: (0, 0)),
            pl.BlockSpec((1, H), lambda i: (0, 0)),
        ],
        out_specs=pl.BlockSpec((bn, H), lambda i: (i, 0)),
        out_shape=jax.ShapeDtypeStruct((N_NODES, H), jnp.float32),
    )(x, aggs, wa, wb, b1, w2, b2)


def kernel(x, adj, inv, W_m1, b_m1, W_m2, b_m2, W_u1, b_u1, W_u2, b_u2):
    adj = adj.astype(jnp.int32)
    src = adj[0].reshape(NW, NCHUNK, CHUNK)
    dst = adj[1].reshape(NW, NCHUNK, CHUNK)

    p, q = _pq(x, W_m1[:H], W_m1[H : 2 * H], b_m1.reshape(1, H))
    t = _gather_add_kernel()(p, q, dst, src)
    m = _msg(t, inv, W_m1[2 * H :], W_m2, b_m2.reshape(1, H))
    aggs = _scatter_add_kernel()(m, dst)
    return _upd(
        x,
        aggs,
        W_u1[:H],
        W_u1[H:],
        b_u1.reshape(1, H),
        W_u2,
        b_u2.reshape(1, H),
    )


# speedup vs baseline: 1.1571x; 1.1571x over previous
import functools

import jax
import jax.numpy as jnp
from jax import lax
from jax.experimental import pallas as pl
from jax.experimental.pallas import tpu as pltpu
from jax.experimental.pallas import tpu_sc as plsc

N_NODES = 10000
N_EDGES = 320000
H = 128
D_INV = 16

NC = 2
NS = 16
NW = NC * NS

EPW = N_EDGES // NW
CHUNK = 80
NCHUNK = EPW // CHUNK
STRIPE = 640
ZROWS = 40

@functools.cache
def _sc_mesh():
    return plsc.VectorSubcoreMesh(
        core_axis_name="c", subcore_axis_name="s", num_cores=NC, num_subcores=NS
    )


def _silu(v):
    return v * (1.0 / (1.0 + jnp.exp(-v)))


def _pack16(v):
    lo = jax.lax.bitcast_convert_type(v[:, : H // 2].astype(jnp.bfloat16), jnp.uint16)
    hi = jax.lax.bitcast_convert_type(v[:, H // 2 :].astype(jnp.bfloat16), jnp.uint16)
    w = lo.astype(jnp.uint32) | (hi.astype(jnp.uint32) << 16)
    return w.astype(jnp.int32)


def _pq_body(x_ref, wa_ref, wb_ref, b_ref, p_ref, q_ref):
    x = x_ref[...]
    p_ref[...] = jnp.dot(x, wa_ref[...], preferred_element_type=jnp.float32) + b_ref[...]
    q_ref[...] = jnp.dot(x, wb_ref[...], preferred_element_type=jnp.float32)


def _pq(x, wa, wb, b):
    bn = 2000
    grid = (N_NODES // bn,)
    return pl.pallas_call(
        _pq_body,
        grid=grid,
        in_specs=[
            pl.BlockSpec((bn, H), lambda i: (i, 0)),
            pl.BlockSpec((H, H), lambda i: (0, 0)),
            pl.BlockSpec((H, H), lambda i: (0, 0)),
            pl.BlockSpec((1, H), lambda i: (0, 0)),
        ],
        out_specs=[
            pl.BlockSpec((bn, H), lambda i: (i, 0)),
            pl.BlockSpec((bn, H), lambda i: (i, 0)),
        ],
        out_shape=[
            jax.ShapeDtypeStruct((N_NODES, H), jnp.float32),
            jax.ShapeDtypeStruct((N_NODES, H), jnp.float32),
        ],
    )(x, wa, wb, b)


@functools.cache
def _gather_add_kernel():
    return pl.kernel(
        _gather_add_body,
        out_type=jax.ShapeDtypeStruct((N_EDGES, H // 2), jnp.int32),
        mesh=_sc_mesh(),
        scratch_types=[
            pltpu.VMEM((NCHUNK, CHUNK), jnp.int32),
            pltpu.VMEM((NCHUNK, CHUNK), jnp.int32),
            pltpu.VMEM((3, CHUNK, H), jnp.float32),
            pltpu.VMEM((3, CHUNK, H), jnp.float32),
            pltpu.VMEM((2, CHUNK, H // 2), jnp.int32),
            pltpu.SemaphoreType.DMA,
            pltpu.SemaphoreType.DMA,
            pltpu.SemaphoreType.DMA,
            pltpu.SemaphoreType.DMA,
            pltpu.SemaphoreType.DMA,
        ],
    )


def _gather_add_body(
    p_hbm, q_hbm, dst_hbm, src_hbm, t_hbm,
    idx_d, idx_s, bufp, bufq, bufo, semg0, semg1, semg2, sems0, sems1,
):
    wid = lax.axis_index("s") * NC + lax.axis_index("c")
    wbase = wid * EPW
    pltpu.sync_copy(dst_hbm.at[wid], idx_d)
    pltpu.sync_copy(src_hbm.at[wid], idx_s)
    semg = (semg0, semg1, semg2)
    sems = (sems0, sems1)

    def g_issue(c, slot):
        pltpu.async_copy(p_hbm.at[idx_d.at[c]], bufp.at[slot], semg[slot])
        pltpu.async_copy(q_hbm.at[idx_s.at[c]], bufq.at[slot], semg[slot])

    def g_wait(slot):
        pltpu.make_async_copy(p_hbm.at[idx_d.at[0]], bufp.at[slot], semg[slot]).wait()
        pltpu.make_async_copy(q_hbm.at[idx_s.at[0]], bufq.at[slot], semg[slot]).wait()

    def s_issue(c, oslot):
        base = pl.multiple_of(wbase + c * CHUNK, 8)
        pltpu.async_copy(bufo.at[oslot], t_hbm.at[pl.ds(base, CHUNK)], sems[oslot])

    def s_wait(oslot):
        pltpu.make_async_copy(bufo.at[oslot], t_hbm.at[pl.ds(0, CHUNK)], sems[oslot]).wait()

    maskhi = jnp.full((16,), 0xFFFF0000, jnp.uint32)

    def add(slot, oslot):
        def addrow(r, carry):
            bc = jax.lax.bitcast_convert_type
            for cc in range(H // 32):
                sl_lo = pl.ds(cc * 16, 16)
                sl_hi = pl.ds(H // 2 + cc * 16, 16)
                lo = bufp[slot, r, sl_lo] + bufq[slot, r, sl_lo]
                hi = bufp[slot, r, sl_hi] + bufq[slot, r, sl_hi]
                w = (bc(lo, jnp.uint32) >> 16) | (bc(hi, jnp.uint32) & maskhi)
                bufo[oslot, r, sl_lo] = bc(w, jnp.int32)
            return carry

        lax.fori_loop(0, CHUNK, addrow, 0, unroll=2)

    def substep(c, slot, oslot, issue_next, wait_store):
        if issue_next:
            g_issue(c + 2, (slot + 2) % 3)
        g_wait(slot)
        if wait_store is None:
            s_wait(oslot)
        elif wait_store is not None and wait_store is not False:
            @pl.when(wait_store)
            def _():
                s_wait(oslot)
        add(slot, oslot)
        s_issue(c, oslot)

    g_issue(0, 0)
    g_issue(1, 1)

    def group(i, carry):
        c0 = i * 6
        for j in range(6):
            ws = (i > 0) if j < 2 else None
            substep(c0 + j, j % 3, j % 2, True, ws)
        return carry

    lax.fori_loop(0, (NCHUNK - 5) // 6, group, 0)
    for c in range(NCHUNK - 5, NCHUNK):
        substep(c, c % 3, c % 2, c + 2 < NCHUNK, None)
    s_wait((NCHUNK - 2) % 2)
    s_wait((NCHUNK - 1) % 2)


def _msg_body(t_ref, inv_ref, wc_ref, w2_ref, b2_ref, m_ref):
    w = t_ref[...]
    t_lo = jax.lax.bitcast_convert_type(w << 16, jnp.float32)
    t_hi = jax.lax.bitcast_convert_type(
        jnp.bitwise_and(w, jnp.int32(-65536)), jnp.float32
    )
    inv = inv_ref[...]
    h_lo = _silu(
        t_lo + jnp.dot(inv, wc_ref[:, : H // 2], preferred_element_type=jnp.float32)
    )
    h_hi = _silu(
        t_hi + jnp.dot(inv, wc_ref[:, H // 2 :], preferred_element_type=jnp.float32)
    )
    m = (
        jnp.dot(h_lo, w2_ref[: H // 2], preferred_element_type=jnp.float32)
        + jnp.dot(h_hi, w2_ref[H // 2 :], preferred_element_type=jnp.float32)
        + b2_ref[...]
    )
    m_ref[...] = _silu(m)


def _msg(t, inv, wc, w2, b2):
    be = 4000
    grid = (N_EDGES // be,)
    return pl.pallas_call(
        _msg_body,
        grid=grid,
        in_specs=[
            pl.BlockSpec((be, H // 2), lambda i: (i, 0)),
            pl.BlockSpec((be, D_INV), lambda i: (i, 0)),
            pl.BlockSpec((D_INV, H), lambda i: (0, 0)),
            pl.BlockSpec((H, H), lambda i: (0, 0)),
            pl.BlockSpec((1, H), lambda i: (0, 0)),
        ],
        out_specs=pl.BlockSpec((be, H), lambda i: (i, 0)),
        out_shape=jax.ShapeDtypeStruct((N_EDGES, H), jnp.float32),
    )(t, inv, wc, w2, b2)


@functools.cache
def _scatter_add_kernel():
    return pl.kernel(
        _scatter_add_body,
        out_type=jax.ShapeDtypeStruct((NC, N_NODES, H), jnp.float32),
        mesh=_sc_mesh(),
        scratch_types=[
            pltpu.VMEM((NCHUNK, CHUNK), jnp.int32),
            pltpu.VMEM((2, CHUNK, H), jnp.float32),
            pltpu.VMEM((ZROWS, H), jnp.float32),
            pltpu.VMEM_SHARED((N_NODES, H), jnp.float32),
            pltpu.SemaphoreType.DMA,
            pltpu.SemaphoreType.DMA,
            pltpu.SemaphoreType.DMA,
            pltpu.SemaphoreType.DMA,
        ],
    )


def _scatter_add_body(
    m_hbm, dst_hbm, out_hbm, idx_d, mbuf, zbuf, agg_sh,
    seml0, seml1, sema0, sema1,
):
    cid = lax.axis_index("c")
    sid = lax.axis_index("s")
    wid = sid * NC + cid
    wbase = wid * EPW

    zero = jnp.zeros((16,), jnp.float32)

    def zrow(r, carry):
        for cc in range(H // 16):
            zbuf[r, pl.ds(cc * 16, 16)] = zero
        return carry

    lax.fori_loop(0, ZROWS, zrow, 0)
    nstripe = jnp.where(sid == NS - 1, (N_NODES - (NS - 1) * STRIPE) // ZROWS, STRIPE // ZROWS)

    def zcopy(j, carry):
        pltpu.sync_copy(zbuf, agg_sh.at[pl.ds(pl.multiple_of(sid * STRIPE + j * ZROWS, 8), ZROWS)])
        return carry

    lax.fori_loop(0, nstripe, zcopy, 0)
    pltpu.sync_copy(dst_hbm.at[wid], idx_d)
    plsc.subcore_barrier()

    seml = (seml0, seml1)
    sema = (sema0, sema1)

    def m_issue(c, slot):
        base = pl.multiple_of(wbase + c * CHUNK, 8)
        pltpu.async_copy(m_hbm.at[pl.ds(base, CHUNK)], mbuf.at[slot], seml[slot])

    def m_wait(slot):
        pltpu.make_async_copy(m_hbm.at[pl.ds(0, CHUNK)], mbuf.at[slot], seml[slot]).wait()

    def a_issue(c, slot):
        pltpu.async_copy(mbuf.at[slot], agg_sh.at[idx_d.at[c]], sema[slot], add=True)

    def a_wait(slot):
        pltpu.make_async_copy(mbuf.at[slot], agg_sh.at[idx_d.at[0]], sema[slot]).wait()

    m_issue(0, 0)

    def pair(i, carry):
        c0 = i * 2

        @pl.when(i > 0)
        def _():
            a_wait(1)

        m_issue(c0 + 1, 1)
        m_wait(0)
        a_issue(c0, 0)
        m_wait(1)
        a_wait(0)
        m_issue(c0 + 2, 0)
        a_issue(c0 + 1, 1)
        return carry

    lax.fori_loop(0, (NCHUNK - 1) // 2, pair, 0)
    a_wait(1)
    m_wait(0)
    a_issue(NCHUNK - 1, 0)
    a_wait(0)
    plsc.subcore_barrier()

    def ocopy(j, carry):
        r0 = pl.multiple_of(sid * STRIPE + j * ZROWS, 8)
        pltpu.sync_copy(agg_sh.at[pl.ds(r0, ZROWS)], zbuf)
        pltpu.sync_copy(zbuf, out_hbm.at[cid, pl.ds(r0, ZROWS)])
        return carry

    lax.fori_loop(0, nstripe, ocopy, 0)


def _upd_body(x_ref, agg_ref, wa_ref, wb_ref, b1_ref, w2_ref, b2_ref, o_ref):
    x = x_ref[...]
    agg = agg_ref[0] + agg_ref[1]
    u = _silu(
        jnp.dot(x, wa_ref[...], preferred_element_type=jnp.float32)
        + jnp.dot(agg, wb_ref[...], preferred_element_type=jnp.float32)
        + b1_ref[...]
    )
    o_ref[...] = x + jnp.dot(u, w2_ref[...], preferred_element_type=jnp.float32) + b2_ref[...]


def _upd(x, aggs, wa, wb, b1, w2, b2):
    bn = 2000
    grid = (N_NODES // bn,)
    return pl.pallas_call(
        _upd_body,
        grid=grid,
        in_specs=[
            pl.BlockSpec((bn, H), lambda i: (i, 0)),
            pl.BlockSpec((NC, bn, H), lambda i: (0, i, 0)),
            pl.BlockSpec((H, H), lambda i: (0, 0)),
            pl.BlockSpec((H, H), lambda i: (0, 0)),
            pl.BlockSpec((1, H), lambda i: (0, 0)),
            pl.BlockSpec((H, H), lambda i: (0, 0)),
            pl.BlockSpec((1, H), lambda i: (0, 0)),
        ],
        out_specs=pl.BlockSpec((bn, H), lambda i: (i, 0)),
        out_shape=jax.ShapeDtypeStruct((N_NODES, H), jnp.float32),
    )(x, aggs, wa, wb, b1, w2, b2)


def kernel(x, adj, inv, W_m1, b_m1, W_m2, b_m2, W_u1, b_u1, W_u2, b_u2):
    adj = adj.astype(jnp.int32)
    src = adj[0].reshape(NW, NCHUNK, CHUNK)
    dst = adj[1].reshape(NW, NCHUNK, CHUNK)

    p, q = _pq(x, W_m1[:H], W_m1[H : 2 * H], b_m1.reshape(1, H))
    t = _gather_add_kernel()(p, q, dst, src)
    m = _msg(t, inv, W_m1[2 * H :], W_m2, b_m2.reshape(1, H))
    aggs = _scatter_add_kernel()(m, dst)
    return _upd(
        x,
        aggs,
        W_u1[:H],
        W_u1[H:],
        b_u1.reshape(1, H),
        W_u2,
        b_u2.reshape(1, H),
    )


# speedup vs baseline: 1.2103x; 1.0459x over previous
import functools

import jax
import jax.numpy as jnp
from jax import lax
from jax.experimental import pallas as pl
from jax.experimental.pallas import tpu as pltpu
from jax.experimental.pallas import tpu_sc as plsc

N_NODES = 10000
N_EDGES = 320000
H = 128
D_INV = 16

NC = 2
NS = 16
NW = NC * NS

EPW = N_EDGES // NW
CHUNK = 80
NCHUNK = EPW // CHUNK
STRIPE = 640
ZROWS = 40

@functools.cache
def _sc_mesh():
    return plsc.VectorSubcoreMesh(
        core_axis_name="c", subcore_axis_name="s", num_cores=NC, num_subcores=NS
    )


def _silu(v):
    return v * (1.0 / (1.0 + jnp.exp(-v)))


def _pack16(v):
    lo = jax.lax.bitcast_convert_type(v[:, : H // 2].astype(jnp.bfloat16), jnp.uint16)
    hi = jax.lax.bitcast_convert_type(v[:, H // 2 :].astype(jnp.bfloat16), jnp.uint16)
    w = lo.astype(jnp.uint32) | (hi.astype(jnp.uint32) << 16)
    return w.astype(jnp.int32)


def _pq_body(x_ref, wa_ref, wb_ref, b_ref, p_ref, q_ref):
    x = x_ref[...]
    p_ref[...] = jnp.dot(x, wa_ref[...], preferred_element_type=jnp.float32) + b_ref[...]
    q_ref[...] = jnp.dot(x, wb_ref[...], preferred_element_type=jnp.float32)


def _pq(x, wa, wb, b):
    bn = 2000
    grid = (N_NODES // bn,)
    return pl.pallas_call(
        _pq_body,
        grid=grid,
        in_specs=[
            pl.BlockSpec((bn, H), lambda i: (i, 0)),
            pl.BlockSpec((H, H), lambda i: (0, 0)),
            pl.BlockSpec((H, H), lambda i: (0, 0)),
            pl.BlockSpec((1, H), lambda i: (0, 0)),
        ],
        out_specs=[
            pl.BlockSpec((bn, H), lambda i: (i, 0)),
            pl.BlockSpec((bn, H), lambda i: (i, 0)),
        ],
        out_shape=[
            jax.ShapeDtypeStruct((N_NODES, H), jnp.float32),
            jax.ShapeDtypeStruct((N_NODES, H), jnp.float32),
        ],
    )(x, wa, wb, b)


@functools.cache
def _gather_add_kernel():
    return pl.kernel(
        _gather_add_body,
        out_type=jax.ShapeDtypeStruct((N_EDGES, H // 2), jnp.int32),
        mesh=_sc_mesh(),
        scratch_types=[
            pltpu.VMEM((NCHUNK, CHUNK), jnp.int32),
            pltpu.VMEM((NCHUNK, CHUNK), jnp.int32),
            pltpu.VMEM((3, CHUNK, H), jnp.float32),
            pltpu.VMEM((3, CHUNK, H), jnp.float32),
            pltpu.VMEM((2, CHUNK, H // 2), jnp.int32),
            pltpu.SemaphoreType.DMA,
            pltpu.SemaphoreType.DMA,
            pltpu.SemaphoreType.DMA,
            pltpu.SemaphoreType.DMA,
            pltpu.SemaphoreType.DMA,
        ],
    )


def _gather_add_body(
    p_hbm, q_hbm, dst_hbm, src_hbm, t_hbm,
    idx_d, idx_s, bufp, bufq, bufo, semg0, semg1, semg2, sems0, sems1,
):
    wid = lax.axis_index("s") * NC + lax.axis_index("c")
    wbase = wid * EPW
    pltpu.sync_copy(dst_hbm.at[wid], idx_d)
    pltpu.sync_copy(src_hbm.at[wid], idx_s)
    semg = (semg0, semg1, semg2)
    sems = (sems0, sems1)

    def g_issue(c, slot):
        pltpu.async_copy(p_hbm.at[idx_d.at[c]], bufp.at[slot], semg[slot])
        pltpu.async_copy(q_hbm.at[idx_s.at[c]], bufq.at[slot], semg[slot])

    def g_wait(slot):
        pltpu.make_async_copy(p_hbm.at[idx_d.at[0]], bufp.at[slot], semg[slot]).wait()
        pltpu.make_async_copy(q_hbm.at[idx_s.at[0]], bufq.at[slot], semg[slot]).wait()

    def s_issue(c, oslot):
        base = pl.multiple_of(wbase + c * CHUNK, 8)
        pltpu.async_copy(bufo.at[oslot], t_hbm.at[pl.ds(base, CHUNK)], sems[oslot])

    def s_wait(oslot):
        pltpu.make_async_copy(bufo.at[oslot], t_hbm.at[pl.ds(0, CHUNK)], sems[oslot]).wait()

    maskhi = jnp.full((16,), 0xFFFF0000, jnp.uint32)

    def add(slot, oslot):
        def addrow(r, carry):
            bc = jax.lax.bitcast_convert_type
            for cc in range(H // 32):
                sl_lo = pl.ds(cc * 16, 16)
                sl_hi = pl.ds(H // 2 + cc * 16, 16)
                lo = bufp[slot, r, sl_lo] + bufq[slot, r, sl_lo]
                hi = bufp[slot, r, sl_hi] + bufq[slot, r, sl_hi]
                w = (bc(lo, jnp.uint32) >> 16) | (bc(hi, jnp.uint32) & maskhi)
                bufo[oslot, r, sl_lo] = bc(w, jnp.int32)
            return carry

        lax.fori_loop(0, CHUNK, addrow, 0, unroll=2)

    def substep(c, slot, oslot, issue_next, wait_store):
        if issue_next:
            g_issue(c + 2, (slot + 2) % 3)
        g_wait(slot)
        if wait_store is None:
            s_wait(oslot)
        elif wait_store is not None and wait_store is not False:
            @pl.when(wait_store)
            def _():
                s_wait(oslot)
        add(slot, oslot)
        s_issue(c, oslot)

    g_issue(0, 0)
    g_issue(1, 1)

    def group(i, carry):
        c0 = i * 6
        for j in range(6):
            ws = (i > 0) if j < 2 else None
            substep(c0 + j, j % 3, j % 2, True, ws)
        return carry

    lax.fori_loop(0, (NCHUNK - 5) // 6, group, 0)
    for c in range(NCHUNK - 5, NCHUNK):
        substep(c, c % 3, c % 2, c + 2 < NCHUNK, None)
    s_wait((NCHUNK - 2) % 2)
    s_wait((NCHUNK - 1) % 2)


def _msg_body(t_ref, inv_ref, wc_ref, w2_ref, b2_ref, m_ref):
    w = t_ref[...]
    t_lo = jax.lax.bitcast_convert_type(w << 16, jnp.float32)
    t_hi = jax.lax.bitcast_convert_type(
        jnp.bitwise_and(w, jnp.int32(-65536)), jnp.float32
    )
    inv = inv_ref[...]
    h_lo = _silu(
        t_lo + jnp.dot(inv, wc_ref[:, : H // 2], preferred_element_type=jnp.float32)
    )
    h_hi = _silu(
        t_hi + jnp.dot(inv, wc_ref[:, H // 2 :], preferred_element_type=jnp.float32)
    )
    m = (
        jnp.dot(h_lo, w2_ref[: H // 2], preferred_element_type=jnp.float32)
        + jnp.dot(h_hi, w2_ref[H // 2 :], preferred_element_type=jnp.float32)
        + b2_ref[...]
    )
    m_ref[...] = _silu(m)


def _msg(t, inv, wc, w2, b2):
    be = 8000
    grid = (N_EDGES // be,)
    return pl.pallas_call(
        _msg_body,
        grid=grid,
        in_specs=[
            pl.BlockSpec((be, H // 2), lambda i: (i, 0)),
            pl.BlockSpec((be, D_INV), lambda i: (i, 0)),
            pl.BlockSpec((D_INV, H), lambda i: (0, 0)),
            pl.BlockSpec((H, H), lambda i: (0, 0)),
            pl.BlockSpec((1, H), lambda i: (0, 0)),
        ],
        out_specs=pl.BlockSpec((be, H), lambda i: (i, 0)),
        out_shape=jax.ShapeDtypeStruct((N_EDGES, H), jnp.float32),
    )(t, inv, wc, w2, b2)


@functools.cache
def _scatter_add_kernel():
    return pl.kernel(
        _scatter_add_body,
        out_type=jax.ShapeDtypeStruct((NC, N_NODES, H), jnp.float32),
        mesh=_sc_mesh(),
        scratch_types=[
            pltpu.VMEM((NCHUNK, CHUNK), jnp.int32),
            pltpu.VMEM((2, CHUNK, H), jnp.float32),
            pltpu.VMEM((ZROWS, H), jnp.float32),
            pltpu.VMEM_SHARED((N_NODES, H), jnp.float32),
            pltpu.SemaphoreType.DMA,
            pltpu.SemaphoreType.DMA,
            pltpu.SemaphoreType.DMA,
            pltpu.SemaphoreType.DMA,
        ],
    )


def _scatter_add_body(
    m_hbm, dst_hbm, out_hbm, idx_d, mbuf, zbuf, agg_sh,
    seml0, seml1, sema0, sema1,
):
    cid = lax.axis_index("c")
    sid = lax.axis_index("s")
    wid = sid * NC + cid
    wbase = wid * EPW

    zero = jnp.zeros((16,), jnp.float32)

    def zrow(r, carry):
        for cc in range(H // 16):
            zbuf[r, pl.ds(cc * 16, 16)] = zero
        return carry

    lax.fori_loop(0, ZROWS, zrow, 0)
    nstripe = jnp.where(sid == NS - 1, (N_NODES - (NS - 1) * STRIPE) // ZROWS, STRIPE // ZROWS)

    def zcopy(j, carry):
        pltpu.sync_copy(zbuf, agg_sh.at[pl.ds(pl.multiple_of(sid * STRIPE + j * ZROWS, 8), ZROWS)])
        return carry

    lax.fori_loop(0, nstripe, zcopy, 0)
    pltpu.sync_copy(dst_hbm.at[wid], idx_d)
    plsc.subcore_barrier()

    seml = (seml0, seml1)
    sema = (sema0, sema1)

    def m_issue(c, slot):
        base = pl.multiple_of(wbase + c * CHUNK, 8)
        pltpu.async_copy(m_hbm.at[pl.ds(base, CHUNK)], mbuf.at[slot], seml[slot])

    def m_wait(slot):
        pltpu.make_async_copy(m_hbm.at[pl.ds(0, CHUNK)], mbuf.at[slot], seml[slot]).wait()

    def a_issue(c, slot):
        pltpu.async_copy(mbuf.at[slot], agg_sh.at[idx_d.at[c]], sema[slot], add=True)

    def a_wait(slot):
        pltpu.make_async_copy(mbuf.at[slot], agg_sh.at[idx_d.at[0]], sema[slot]).wait()

    m_issue(0, 0)

    def pair(i, carry):
        c0 = i * 2

        @pl.when(i > 0)
        def _():
            a_wait(1)

        m_issue(c0 + 1, 1)
        m_wait(0)
        a_issue(c0, 0)
        m_wait(1)
        a_wait(0)
        m_issue(c0 + 2, 0)
        a_issue(c0 + 1, 1)
        return carry

    lax.fori_loop(0, (NCHUNK - 1) // 2, pair, 0)
    a_wait(1)
    m_wait(0)
    a_issue(NCHUNK - 1, 0)
    a_wait(0)
    plsc.subcore_barrier()

    def ocopy(j, carry):
        r0 = pl.multiple_of(sid * STRIPE + j * ZROWS, 8)
        pltpu.sync_copy(agg_sh.at[pl.ds(r0, ZROWS)], zbuf)
        pltpu.sync_copy(zbuf, out_hbm.at[cid, pl.ds(r0, ZROWS)])
        return carry

    lax.fori_loop(0, nstripe, ocopy, 0)


def _upd_body(x_ref, agg_ref, wa_ref, wb_ref, b1_ref, w2_ref, b2_ref, o_ref):
    x = x_ref[...]
    agg = agg_ref[0] + agg_ref[1]
    u = _silu(
        jnp.dot(x, wa_ref[...], preferred_element_type=jnp.float32)
        + jnp.dot(agg, wb_ref[...], preferred_element_type=jnp.float32)
        + b1_ref[...]
    )
    o_ref[...] = x + jnp.dot(u, w2_ref[...], preferred_element_type=jnp.float32) + b2_ref[...]


def _upd(x, aggs, wa, wb, b1, w2, b2):
    bn = 2000
    grid = (N_NODES // bn,)
    return pl.pallas_call(
        _upd_body,
        grid=grid,
        in_specs=[
            pl.BlockSpec((bn, H), lambda i: (i, 0)),
            pl.BlockSpec((NC, bn, H), lambda i: (0, i, 0)),
            pl.BlockSpec((H, H), lambda i: (0, 0)),
            pl.BlockSpec((H, H), lambda i: (0, 0)),
            pl.BlockSpec((1, H), lambda i: (0, 0)),
            pl.BlockSpec((H, H), lambda i: (0, 0)),
            pl.BlockSpec((1, H), lambda i: (0, 0)),
        ],
        out_specs=pl.BlockSpec((bn, H), lambda i: (i, 0)),
        out_shape=jax.ShapeDtypeStruct((N_NODES, H), jnp.float32),
    )(x, aggs, wa, wb, b1, w2, b2)


def kernel(x, adj, inv, W_m1, b_m1, W_m2, b_m2, W_u1, b_u1, W_u2, b_u2):
    adj = adj.astype(jnp.int32)
    src = adj[0].reshape(NW, NCHUNK, CHUNK)
    dst = adj[1].reshape(NW, NCHUNK, CHUNK)

    p, q = _pq(x, W_m1[:H], W_m1[H : 2 * H], b_m1.reshape(1, H))
    t = _gather_add_kernel()(p, q, dst, src)
    m = _msg(t, inv, W_m1[2 * H :], W_m2, b_m2.reshape(1, H))
    aggs = _scatter_add_kernel()(m, dst)
    return _upd(
        x,
        aggs,
        W_u1[:H],
        W_u1[H:],
        b_u1.reshape(1, H),
        W_u2,
        b_u2.reshape(1, H),
    )


# speedup vs baseline: 1.2157x; 1.0045x over previous
import functools

import jax
import jax.numpy as jnp
from jax import lax
from jax.experimental import pallas as pl
from jax.experimental.pallas import tpu as pltpu
from jax.experimental.pallas import tpu_sc as plsc

N_NODES = 10000
N_EDGES = 320000
H = 128
D_INV = 16

NC = 2
NS = 16
NW = NC * NS

EPW = N_EDGES // NW
CHUNK = 80
NCHUNK = EPW // CHUNK
STRIPE = 640
ZROWS = 40

@functools.cache
def _sc_mesh():
    return plsc.VectorSubcoreMesh(
        core_axis_name="c", subcore_axis_name="s", num_cores=NC, num_subcores=NS
    )


def _silu(v):
    return v * (1.0 / (1.0 + jnp.exp(-v)))


def _pack16(v):
    lo = jax.lax.bitcast_convert_type(v[:, : H // 2].astype(jnp.bfloat16), jnp.uint16)
    hi = jax.lax.bitcast_convert_type(v[:, H // 2 :].astype(jnp.bfloat16), jnp.uint16)
    w = lo.astype(jnp.uint32) | (hi.astype(jnp.uint32) << 16)
    return w.astype(jnp.int32)


def _pq_body(x_ref, wa_ref, wb_ref, b_ref, p_ref, q_ref):
    x = x_ref[...]
    p_ref[...] = jnp.dot(x, wa_ref[...], preferred_element_type=jnp.float32) + b_ref[...]
    q_ref[...] = jnp.dot(x, wb_ref[...], preferred_element_type=jnp.float32)


def _pq(x, wa, wb, b):
    bn = 5000
    grid = (N_NODES // bn,)
    return pl.pallas_call(
        _pq_body,
        grid=grid,
        in_specs=[
            pl.BlockSpec((bn, H), lambda i: (i, 0)),
            pl.BlockSpec((H, H), lambda i: (0, 0)),
            pl.BlockSpec((H, H), lambda i: (0, 0)),
            pl.BlockSpec((1, H), lambda i: (0, 0)),
        ],
        out_specs=[
            pl.BlockSpec((bn, H), lambda i: (i, 0)),
            pl.BlockSpec((bn, H), lambda i: (i, 0)),
        ],
        out_shape=[
            jax.ShapeDtypeStruct((N_NODES, H), jnp.float32),
            jax.ShapeDtypeStruct((N_NODES, H), jnp.float32),
        ],
    )(x, wa, wb, b)


@functools.cache
def _gather_add_kernel():
    return pl.kernel(
        _gather_add_body,
        out_type=jax.ShapeDtypeStruct((N_EDGES, H // 2), jnp.int32),
        mesh=_sc_mesh(),
        scratch_types=[
            pltpu.VMEM((NCHUNK, CHUNK), jnp.int32),
            pltpu.VMEM((NCHUNK, CHUNK), jnp.int32),
            pltpu.VMEM((3, CHUNK, H), jnp.float32),
            pltpu.VMEM((3, CHUNK, H), jnp.float32),
            pltpu.VMEM((2, CHUNK, H // 2), jnp.int32),
            pltpu.SemaphoreType.DMA,
            pltpu.SemaphoreType.DMA,
            pltpu.SemaphoreType.DMA,
            pltpu.SemaphoreType.DMA,
            pltpu.SemaphoreType.DMA,
        ],
    )


def _gather_add_body(
    p_hbm, q_hbm, dst_hbm, src_hbm, t_hbm,
    idx_d, idx_s, bufp, bufq, bufo, semg0, semg1, semg2, sems0, sems1,
):
    wid = lax.axis_index("s") * NC + lax.axis_index("c")
    wbase = wid * EPW
    pltpu.sync_copy(dst_hbm.at[wid], idx_d)
    pltpu.sync_copy(src_hbm.at[wid], idx_s)
    semg = (semg0, semg1, semg2)
    sems = (sems0, sems1)

    def g_issue(c, slot):
        pltpu.async_copy(p_hbm.at[idx_d.at[c]], bufp.at[slot], semg[slot])
        pltpu.async_copy(q_hbm.at[idx_s.at[c]], bufq.at[slot], semg[slot])

    def g_wait(slot):
        pltpu.make_async_copy(p_hbm.at[idx_d.at[0]], bufp.at[slot], semg[slot]).wait()
        pltpu.make_async_copy(q_hbm.at[idx_s.at[0]], bufq.at[slot], semg[slot]).wait()

    def s_issue(c, oslot):
        base = pl.multiple_of(wbase + c * CHUNK, 8)
        pltpu.async_copy(bufo.at[oslot], t_hbm.at[pl.ds(base, CHUNK)], sems[oslot])

    def s_wait(oslot):
        pltpu.make_async_copy(bufo.at[oslot], t_hbm.at[pl.ds(0, CHUNK)], sems[oslot]).wait()

    maskhi = jnp.full((16,), 0xFFFF0000, jnp.uint32)

    def add(slot, oslot):
        def addrow(r, carry):
            bc = jax.lax.bitcast_convert_type
            for cc in range(H // 32):
                sl_lo = pl.ds(cc * 16, 16)
                sl_hi = pl.ds(H // 2 + cc * 16, 16)
                lo = bufp[slot, r, sl_lo] + bufq[slot, r, sl_lo]
                hi = bufp[slot, r, sl_hi] + bufq[slot, r, sl_hi]
                w = (bc(lo, jnp.uint32) >> 16) | (bc(hi, jnp.uint32) & maskhi)
                bufo[oslot, r, sl_lo] = bc(w, jnp.int32)
            return carry

        lax.fori_loop(0, CHUNK, addrow, 0, unroll=2)

    def substep(c, slot, oslot, issue_next, wait_store):
        if issue_next:
            g_issue(c + 2, (slot + 2) % 3)
        g_wait(slot)
        if wait_store is None:
            s_wait(oslot)
        elif wait_store is not None and wait_store is not False:
            @pl.when(wait_store)
            def _():
                s_wait(oslot)
        add(slot, oslot)
        s_issue(c, oslot)

    g_issue(0, 0)
    g_issue(1, 1)

    def group(i, carry):
        c0 = i * 6
        for j in range(6):
            ws = (i > 0) if j < 2 else None
            substep(c0 + j, j % 3, j % 2, True, ws)
        return carry

    lax.fori_loop(0, (NCHUNK - 5) // 6, group, 0)
    for c in range(NCHUNK - 5, NCHUNK):
        substep(c, c % 3, c % 2, c + 2 < NCHUNK, None)
    s_wait((NCHUNK - 2) % 2)
    s_wait((NCHUNK - 1) % 2)


def _msg_body(t_ref, inv_ref, wc_ref, w2_ref, b2_ref, m_ref):
    w = t_ref[...]
    t_lo = jax.lax.bitcast_convert_type(w << 16, jnp.float32)
    t_hi = jax.lax.bitcast_convert_type(
        jnp.bitwise_and(w, jnp.int32(-65536)), jnp.float32
    )
    inv = inv_ref[...]
    h_lo = _silu(
        t_lo + jnp.dot(inv, wc_ref[:, : H // 2], preferred_element_type=jnp.float32)
    )
    h_hi = _silu(
        t_hi + jnp.dot(inv, wc_ref[:, H // 2 :], preferred_element_type=jnp.float32)
    )
    m = (
        jnp.dot(h_lo, w2_ref[: H // 2], preferred_element_type=jnp.float32)
        + jnp.dot(h_hi, w2_ref[H // 2 :], preferred_element_type=jnp.float32)
        + b2_ref[...]
    )
    m_ref[...] = _silu(m)


def _msg(t, inv, wc, w2, b2):
    be = 8000
    grid = (N_EDGES // be,)
    return pl.pallas_call(
        _msg_body,
        grid=grid,
        in_specs=[
            pl.BlockSpec((be, H // 2), lambda i: (i, 0)),
            pl.BlockSpec((be, D_INV), lambda i: (i, 0)),
            pl.BlockSpec((D_INV, H), lambda i: (0, 0)),
            pl.BlockSpec((H, H), lambda i: (0, 0)),
            pl.BlockSpec((1, H), lambda i: (0, 0)),
        ],
        out_specs=pl.BlockSpec((be, H), lambda i: (i, 0)),
        out_shape=jax.ShapeDtypeStruct((N_EDGES, H), jnp.float32),
    )(t, inv, wc, w2, b2)


@functools.cache
def _scatter_add_kernel():
    return pl.kernel(
        _scatter_add_body,
        out_type=jax.ShapeDtypeStruct((NC, N_NODES, H), jnp.float32),
        mesh=_sc_mesh(),
        scratch_types=[
            pltpu.VMEM((NCHUNK, CHUNK), jnp.int32),
            pltpu.VMEM((2, CHUNK, H), jnp.float32),
            pltpu.VMEM((ZROWS, H), jnp.float32),
            pltpu.VMEM_SHARED((N_NODES, H), jnp.float32),
            pltpu.SemaphoreType.DMA,
            pltpu.SemaphoreType.DMA,
            pltpu.SemaphoreType.DMA,
            pltpu.SemaphoreType.DMA,
        ],
    )


def _scatter_add_body(
    m_hbm, dst_hbm, out_hbm, idx_d, mbuf, zbuf, agg_sh,
    seml0, seml1, sema0, sema1,
):
    cid = lax.axis_index("c")
    sid = lax.axis_index("s")
    wid = sid * NC + cid
    wbase = wid * EPW

    zero = jnp.zeros((16,), jnp.float32)

    def zrow(r, carry):
        for cc in range(H // 16):
            zbuf[r, pl.ds(cc * 16, 16)] = zero
        return carry

    lax.fori_loop(0, ZROWS, zrow, 0)
    nstripe = jnp.where(sid == NS - 1, (N_NODES - (NS - 1) * STRIPE) // ZROWS, STRIPE // ZROWS)

    def zcopy(j, carry):
        pltpu.sync_copy(zbuf, agg_sh.at[pl.ds(pl.multiple_of(sid * STRIPE + j * ZROWS, 8), ZROWS)])
        return carry

    lax.fori_loop(0, nstripe, zcopy, 0)
    pltpu.sync_copy(dst_hbm.at[wid], idx_d)
    plsc.subcore_barrier()

    seml = (seml0, seml1)
    sema = (sema0, sema1)

    def m_issue(c, slot):
        base = pl.multiple_of(wbase + c * CHUNK, 8)
        pltpu.async_copy(m_hbm.at[pl.ds(base, CHUNK)], mbuf.at[slot], seml[slot])

    def m_wait(slot):
        pltpu.make_async_copy(m_hbm.at[pl.ds(0, CHUNK)], mbuf.at[slot], seml[slot]).wait()

    def a_issue(c, slot):
        pltpu.async_copy(mbuf.at[slot], agg_sh.at[idx_d.at[c]], sema[slot], add=True)

    def a_wait(slot):
        pltpu.make_async_copy(mbuf.at[slot], agg_sh.at[idx_d.at[0]], sema[slot]).wait()

    m_issue(0, 0)

    def pair(i, carry):
        c0 = i * 2

        @pl.when(i > 0)
        def _():
            a_wait(1)

        m_issue(c0 + 1, 1)
        m_wait(0)
        a_issue(c0, 0)
        m_wait(1)
        a_wait(0)
        m_issue(c0 + 2, 0)
        a_issue(c0 + 1, 1)
        return carry

    lax.fori_loop(0, (NCHUNK - 1) // 2, pair, 0)
    a_wait(1)
    m_wait(0)
    a_issue(NCHUNK - 1, 0)
    a_wait(0)
    plsc.subcore_barrier()

    def ocopy(j, carry):
        r0 = pl.multiple_of(sid * STRIPE + j * ZROWS, 8)
        pltpu.sync_copy(agg_sh.at[pl.ds(r0, ZROWS)], zbuf)
        pltpu.sync_copy(zbuf, out_hbm.at[cid, pl.ds(r0, ZROWS)])
        return carry

    lax.fori_loop(0, nstripe, ocopy, 0)


def _upd_body(x_ref, agg_ref, wa_ref, wb_ref, b1_ref, w2_ref, b2_ref, o_ref):
    x = x_ref[...]
    agg = agg_ref[0] + agg_ref[1]
    u = _silu(
        jnp.dot(x, wa_ref[...], preferred_element_type=jnp.float32)
        + jnp.dot(agg, wb_ref[...], preferred_element_type=jnp.float32)
        + b1_ref[...]
    )
    o_ref[...] = x + jnp.dot(u, w2_ref[...], preferred_element_type=jnp.float32) + b2_ref[...]


def _upd(x, aggs, wa, wb, b1, w2, b2):
    bn = 5000
    grid = (N_NODES // bn,)
    return pl.pallas_call(
        _upd_body,
        grid=grid,
        in_specs=[
            pl.BlockSpec((bn, H), lambda i: (i, 0)),
            pl.BlockSpec((NC, bn, H), lambda i: (0, i, 0)),
            pl.BlockSpec((H, H), lambda i: (0, 0)),
            pl.BlockSpec((H, H), lambda i: (0, 0)),
            pl.BlockSpec((1, H), lambda i: (0, 0)),
            pl.BlockSpec((H, H), lambda i: (0, 0)),
            pl.BlockSpec((1, H), lambda i: (0, 0)),
        ],
        out_specs=pl.BlockSpec((bn, H), lambda i: (i, 0)),
        out_shape=jax.ShapeDtypeStruct((N_NODES, H), jnp.float32),
    )(x, aggs, wa, wb, b1, w2, b2)


def kernel(x, adj, inv, W_m1, b_m1, W_m2, b_m2, W_u1, b_u1, W_u2, b_u2):
    adj = adj.astype(jnp.int32)
    src = adj[0].reshape(NW, NCHUNK, CHUNK)
    dst = adj[1].reshape(NW, NCHUNK, CHUNK)

    p, q = _pq(x, W_m1[:H], W_m1[H : 2 * H], b_m1.reshape(1, H))
    t = _gather_add_kernel()(p, q, dst, src)
    m = _msg(t, inv, W_m1[2 * H :], W_m2, b_m2.reshape(1, H))
    aggs = _scatter_add_kernel()(m, dst)
    return _upd(
        x,
        aggs,
        W_u1[:H],
        W_u1[H:],
        b_u1.reshape(1, H),
        W_u2,
        b_u2.reshape(1, H),
    )
